# Initial kernel scaffold; baseline (speedup 1.0000x reference)
import jax
import jax.numpy as jnp
from jax.experimental import pallas as pl


def kernel(msg, edge_index):
    raise NotImplementedError("write your pallas kernel here")


# speedup vs baseline: 7.9413x; 7.9413x over previous
import functools

import jax
import jax.numpy as jnp
from jax import lax
from jax.experimental import pallas as pl
from jax.experimental.pallas import tpu as pltpu
from jax.experimental.pallas import tpu_sc as plsc

NUM_NODES = 10000
NPAD = 10240
FEAT = 128
BLK = 128
NUM_CORES = 2
NUM_SUBCORES = 16
NUM_TILES = NUM_CORES * NUM_SUBCORES
ROWS_PER_TILE = NPAD // NUM_SUBCORES


def _sc_partials(msg, dst):
    num_edges = msg.shape[0]
    assert num_edges % BLK == 0
    num_blocks = num_edges // BLK
    bpt = -(-num_blocks // NUM_TILES)

    mesh = plsc.VectorSubcoreMesh(core_axis_name="c", subcore_axis_name="s")

    @functools.partial(
        pl.kernel,
        mesh=mesh,
        out_type=jax.ShapeDtypeStruct((NUM_CORES, NPAD, FEAT), jnp.float32),
        scratch_types=[
            pltpu.VMEM_SHARED((NPAD, FEAT), jnp.float32),
            pltpu.VMEM((2, BLK, FEAT), jnp.float32),
            pltpu.VMEM((2, BLK), jnp.int32),
            pltpu.SemaphoreType.DMA((2,)),
            pltpu.SemaphoreType.DMA((2,)),
        ],
    )
    def sc_kernel(msg_hbm, dst_hbm, out_hbm, acc, msg_v, idx_v, msg_sem, idx_sem):
        c = lax.axis_index("c")
        s = lax.axis_index("s")
        w = c * NUM_SUBCORES + s

        @pl.loop(0, BLK)
        def _(r):
            @pl.loop(0, FEAT, step=16)
            def _(f):
                msg_v[0, r, pl.ds(f, 16)] = jnp.zeros((16,), jnp.float32)

        @pl.loop(0, ROWS_PER_TILE // BLK)
        def _(j):
            pltpu.sync_copy(
                msg_v.at[0],
                acc.at[pl.ds(s * ROWS_PER_TILE + j * BLK, BLK)],
            )

        plsc.subcore_barrier()

        base = w * bpt
        nb = jnp.minimum(bpt, num_blocks - base)

        def issue(i, b):
            e0 = (base + i) * BLK
            pltpu.async_copy(dst_hbm.at[pl.ds(e0, BLK)], idx_v.at[b], idx_sem.at[b])
            pltpu.async_copy(msg_hbm.at[pl.ds(e0, BLK)], msg_v.at[b], msg_sem.at[b])

        def wait(i, b):
            e0 = (base + i) * BLK
            pltpu.make_async_copy(dst_hbm.at[pl.ds(e0, BLK)], idx_v.at[b], idx_sem.at[b]).wait()
            pltpu.make_async_copy(msg_hbm.at[pl.ds(e0, BLK)], msg_v.at[b], msg_sem.at[b]).wait()

        issue(0, 0)

        @pl.when(nb > 1)
        def _():
            issue(1, 1)

        def body(j, carry):
            i0 = 2 * j

            @pl.when(i0 < nb)
            def _():
                wait(i0, 0)
                pltpu.sync_copy(msg_v.at[0], acc.at[idx_v.at[0]], add=True)

                @pl.when(i0 + 2 < nb)
                def _():
                    issue(i0 + 2, 0)

            @pl.when(i0 + 1 < nb)
            def _():
                wait(i0 + 1, 1)
                pltpu.sync_copy(msg_v.at[1], acc.at[idx_v.at[1]], add=True)

                @pl.when(i0 + 3 < nb)
                def _():
                    issue(i0 + 3, 1)

            return carry

        lax.fori_loop(0, (nb + 1) // 2, body, 0)

        plsc.subcore_barrier()

        pltpu.sync_copy(
            acc.at[pl.ds(s * ROWS_PER_TILE, ROWS_PER_TILE)],
            out_hbm.at[c, pl.ds(s * ROWS_PER_TILE, ROWS_PER_TILE)],
        )

    return sc_kernel(msg, dst)


def _tc_add(partials):
    def add_body(p_ref, o_ref):
        o_ref[...] = p_ref[0, :NUM_NODES] + p_ref[1, :NUM_NODES]

    return pl.pallas_call(
        add_body,
        out_shape=jax.ShapeDtypeStruct((NUM_NODES, FEAT), jnp.float32),
    )(partials)


def kernel(msg, edge_index):
    dst = edge_index[1].astype(jnp.int32)
    partials = _sc_partials(msg, dst)
    return _tc_add(partials)
